# baseline (speedup 1.0000x reference)
import functools

import jax
import jax.numpy as jnp
from jax import lax
from jax.experimental import pallas as pl
from jax.experimental.pallas import tpu as pltpu
from jax.experimental.pallas import tpu_sc as plsc

_PACK = 16


def _tc_kernel(ea_ref, na_ref, w1bd_ref, b1t_ref, w2_ref, b2_ref, root_ref,
               bias_ref, out_ref, hsum_ref, *, N, HID, IN_C, OUT_C, CH, NC):
    c = pl.program_id(1)
    x = ea_ref[0]
    h = jnp.maximum(
        jnp.dot(x, w1bd_ref[...], preferred_element_type=jnp.float32)
        + b1t_ref[0], 0.0)
    part = jnp.sum(h.reshape(CH // _PACK, _PACK, _PACK * HID), axis=0)

    @pl.when(c == 0)
    def _():
        hsum_ref[...] = part

    @pl.when(c > 0)
    def _():
        hsum_ref[...] = hsum_ref[...] + part

    @pl.when(c == NC - 1)
    def _():
        KW = _PACK * HID
        A = (lax.broadcasted_iota(jnp.int32, (N, _PACK), 0) // _PACK ==
             lax.broadcasted_iota(jnp.int32, (N, _PACK), 1)).astype(jnp.float32)
        G = jnp.dot(A, hsum_ref[...], preferred_element_type=jnp.float32)
        M = (lax.broadcasted_iota(jnp.int32, (N, KW), 1) // HID ==
             lax.broadcasted_iota(jnp.int32, (N, KW), 0) % _PACK).astype(jnp.float32)
        ws = jnp.dot(G * M, w2_ref[...],
                     preferred_element_type=jnp.float32) + N * b2_ref[0]
        na = na_ref[0]
        KO = IN_C * OUT_C
        R = (lax.broadcasted_iota(jnp.int32, (IN_C, KO), 1) // OUT_C ==
             lax.broadcasted_iota(jnp.int32, (IN_C, KO), 0)).astype(jnp.float32)
        S = (lax.broadcasted_iota(jnp.int32, (KO, OUT_C), 0) % OUT_C ==
             lax.broadcasted_iota(jnp.int32, (KO, OUT_C), 1)).astype(jnp.float32)
        msg = jnp.dot(jnp.dot(na, R, preferred_element_type=jnp.float32) * ws, S,
                      preferred_element_type=jnp.float32)
        out_ref[0] = msg + jnp.dot(na, root_ref[...],
                                   preferred_element_type=jnp.float32) + bias_ref[0]


def _sc_repack(edge_adj, idx_tab, B, N, D_EDGE):
    info = plsc.get_sparse_core_info()
    ncores, nsub = info.num_cores, info.num_subcores
    nw = ncores * nsub
    pairs = B * N
    per_w = pairs // nw
    rows = N * N // _PACK
    ngroups = _PACK * D_EDGE
    mesh = plsc.VectorSubcoreMesh(core_axis_name="c", subcore_axis_name="s")

    @functools.partial(
        pl.kernel, mesh=mesh,
        out_type=jax.ShapeDtypeStruct((B, rows, _PACK * D_EDGE), jnp.float32),
        scratch_types=[
            pltpu.VMEM((N, D_EDGE), jnp.float32),
            pltpu.VMEM((_PACK, _PACK * D_EDGE), jnp.float32),
            pltpu.VMEM((ngroups + 8, 16), jnp.int32),
        ],
        compiler_params=pltpu.CompilerParams(needs_layout_passes=False),
    )
    def repack(ea_hbm, idx_hbm, out_hbm, in_v, out_v, idx_v):
        wid = lax.axis_index("s") * ncores + lax.axis_index("c")
        pltpu.sync_copy(idx_hbm, idx_v)
        dcol = idx_v[ngroups, :]

        def body(t, carry):
            pair = wid * per_w + t
            b = pair // N
            i = pair % N
            pltpu.sync_copy(ea_hbm.at[b, i], in_v)
            for q in range(_PACK):
                for t8 in range(D_EDGE):
                    r = idx_v[q * D_EDGE + t8, :]
                    vals = plsc.load_gather(in_v, [r, dcol])
                    out_v[q, pl.ds(16 * t8, 16)] = vals
            pltpu.sync_copy(out_v, out_hbm.at[b, pl.ds(i * _PACK, _PACK)])
            return carry

        lax.fori_loop(0, per_w, body, 0)

    return repack(edge_adj, idx_tab)


def kernel(node_attr, edge_adj, W1, b1, W2, b2, root, bias):
    B, N, IN_C = node_attr.shape
    D_EDGE = edge_adj.shape[-1]
    HID = W1.shape[1]
    OUT_C = root.shape[1]
    ROWS = N * N // _PACK
    CH = 1024
    NC = ROWS // CH

    import numpy as np
    u = np.arange(16)
    tab = np.zeros((_PACK * D_EDGE + 8, 16), np.int32)
    for q in range(_PACK):
        for t8 in range(D_EDGE):
            f = 128 * q + 16 * t8 + u
            tab[q * D_EDGE + t8] = f // D_EDGE
    tab[_PACK * D_EDGE] = u % D_EDGE
    ea_p = _sc_repack(edge_adj, jnp.asarray(tab), B, N, D_EDGE)

    w1bd = jnp.kron(jnp.eye(_PACK, dtype=W1.dtype), W1)
    b1t = jnp.tile(b1, _PACK).reshape(1, _PACK * HID)
    w2t = jnp.tile(W2, (_PACK, 1))
    b2r = b2.reshape(1, IN_C * OUT_C)
    biasr = bias.reshape(1, OUT_C)

    kern = functools.partial(_tc_kernel, N=N, HID=HID, IN_C=IN_C,
                             OUT_C=OUT_C, CH=CH, NC=NC)
    out = pl.pallas_call(
        kern,
        grid=(B, NC),
        in_specs=[
            pl.BlockSpec((1, CH, _PACK * D_EDGE), lambda b, c: (b, c, 0)),
            pl.BlockSpec((1, N, IN_C), lambda b, c: (b, 0, 0)),
            pl.BlockSpec((_PACK * D_EDGE, _PACK * HID), lambda b, c: (0, 0)),
            pl.BlockSpec((1, _PACK * HID), lambda b, c: (0, 0)),
            pl.BlockSpec((_PACK * HID, IN_C * OUT_C), lambda b, c: (0, 0)),
            pl.BlockSpec((1, IN_C * OUT_C), lambda b, c: (0, 0)),
            pl.BlockSpec((IN_C, OUT_C), lambda b, c: (0, 0)),
            pl.BlockSpec((1, OUT_C), lambda b, c: (0, 0)),
        ],
        out_specs=pl.BlockSpec((1, N, OUT_C), lambda b, c: (b, 0, 0)),
        out_shape=jax.ShapeDtypeStruct((B, N, OUT_C), jnp.float32),
        scratch_shapes=[pltpu.VMEM((_PACK, _PACK * HID), jnp.float32)],
        compiler_params=pltpu.CompilerParams(
            dimension_semantics=("parallel", "arbitrary")),
    )(ea_p, node_attr, w1bd, b1t, w2t, b2r, root, biasr)
    return out


# speedup vs baseline: 1.1134x; 1.1134x over previous
import functools

import jax
import jax.numpy as jnp
from jax import lax
from jax.experimental import pallas as pl
from jax.experimental.pallas import tpu as pltpu
from jax.experimental.pallas import tpu_sc as plsc

_PACK = 16


def _tc_kernel(ea_ref, na_ref, w1bd_ref, b1t_ref, w2_ref, b2_ref, root_ref,
               bias_ref, out_ref, hsum_ref, *, N, HID, IN_C, OUT_C, CH, NC):
    c = pl.program_id(1)
    x = ea_ref[0]
    h = jnp.maximum(
        jnp.dot(x, w1bd_ref[...], preferred_element_type=jnp.float32)
        + b1t_ref[0], 0.0)
    part = jnp.sum(h.reshape(CH // _PACK, _PACK, _PACK * HID), axis=0)

    @pl.when(c == 0)
    def _():
        hsum_ref[...] = part

    @pl.when(c > 0)
    def _():
        hsum_ref[...] = hsum_ref[...] + part

    @pl.when(c == NC - 1)
    def _():
        KW = _PACK * HID
        A = (lax.broadcasted_iota(jnp.int32, (N, _PACK), 0) // _PACK ==
             lax.broadcasted_iota(jnp.int32, (N, _PACK), 1)).astype(jnp.float32)
        G = jnp.dot(A, hsum_ref[...], preferred_element_type=jnp.float32)
        M = (lax.broadcasted_iota(jnp.int32, (N, KW), 1) // HID ==
             lax.broadcasted_iota(jnp.int32, (N, KW), 0) % _PACK).astype(jnp.float32)
        ws = jnp.dot(G * M, w2_ref[...],
                     preferred_element_type=jnp.float32) + N * b2_ref[0]
        na = na_ref[0]
        KO = IN_C * OUT_C
        R = (lax.broadcasted_iota(jnp.int32, (IN_C, KO), 1) // OUT_C ==
             lax.broadcasted_iota(jnp.int32, (IN_C, KO), 0)).astype(jnp.float32)
        S = (lax.broadcasted_iota(jnp.int32, (KO, OUT_C), 0) % OUT_C ==
             lax.broadcasted_iota(jnp.int32, (KO, OUT_C), 1)).astype(jnp.float32)
        msg = jnp.dot(jnp.dot(na, R, preferred_element_type=jnp.float32) * ws, S,
                      preferred_element_type=jnp.float32)
        out_ref[0] = msg + jnp.dot(na, root_ref[...],
                                   preferred_element_type=jnp.float32) + bias_ref[0]


def _sc_repack(edge_adj, idx_tab, B, N, D_EDGE):
    info = plsc.get_sparse_core_info()
    ncores, nsub = info.num_cores, info.num_subcores
    nw = ncores * nsub
    pairs = B * N
    per_w = pairs // nw
    grp = 4
    ngr = per_w // grp
    wpb = N // per_w
    rows = N * N // _PACK
    ngroups = _PACK * D_EDGE
    mesh = plsc.VectorSubcoreMesh(core_axis_name="c", subcore_axis_name="s")

    @functools.partial(
        pl.kernel, mesh=mesh,
        out_type=jax.ShapeDtypeStruct((B, rows, _PACK * D_EDGE), jnp.float32),
        scratch_types=[
            pltpu.VMEM((grp, N, D_EDGE), jnp.float32),
            pltpu.VMEM((grp, N, D_EDGE), jnp.float32),
            pltpu.VMEM((grp * _PACK, _PACK * D_EDGE), jnp.float32),
            pltpu.VMEM((grp * _PACK, _PACK * D_EDGE), jnp.float32),
            pltpu.VMEM((ngroups + 16, 16), jnp.int32),
            pltpu.SemaphoreType.DMA,
            pltpu.SemaphoreType.DMA,
            pltpu.SemaphoreType.DMA,
            pltpu.SemaphoreType.DMA,
        ],
        compiler_params=pltpu.CompilerParams(needs_layout_passes=False, use_tc_tiling_on_sc=False),
    )
    def repack(ea_hbm, idx_hbm, out_hbm, in0, in1, ou0, ou1, idx_v,
               isem0, isem1, osem0, osem1):
        wid = lax.axis_index("s") * ncores + lax.axis_index("c")
        pltpu.sync_copy(idx_hbm, idx_v)
        dcol = idx_v[ngroups, :]
        b = wid // wpb
        ibase = (wid % wpb) * per_w
        ins = (in0, in1)
        ous = (ou0, ou1)
        isems = (isem0, isem1)
        osems = (osem0, osem1)

        def in_src(g):
            return ea_hbm.at[b, pl.ds(ibase + grp * g, grp)]

        def out_dst(g):
            return out_hbm.at[b, pl.ds(_PACK * (ibase + grp * g), grp * _PACK)]

        pltpu.make_async_copy(in_src(0), in0, isem0).start()
        pltpu.make_async_copy(in_src(1), in1, isem1).start()

        def body(h, carry):
            for par in range(2):
                g = 2 * h + par
                iv, ov = ins[par], ous[par]
                pltpu.make_async_copy(in_src(g), iv, isems[par]).wait()

                @pl.when(h > 0)
                def _():
                    pltpu.make_async_copy(ov, out_dst(g), osems[par]).wait()

                for sl in range(grp):
                    svec = idx_v[ngroups + 1 + sl, :]
                    for q in range(_PACK):
                        for t8 in range(D_EDGE):
                            r = idx_v[q * D_EDGE + t8, :]
                            vals = plsc.load_gather(iv, [svec, r, dcol])
                            ov[sl * _PACK + q, pl.ds(16 * t8, 16)] = vals
                pltpu.make_async_copy(ov, out_dst(g), osems[par]).start()

                @pl.when(h + 1 < ngr // 2)
                def _():
                    pltpu.make_async_copy(in_src(g + 2), iv, isems[par]).start()
            return carry

        lax.fori_loop(0, ngr // 2, body, 0)
        pltpu.make_async_copy(ou0, out_dst(ngr - 2), osem0).wait()
        pltpu.make_async_copy(ou1, out_dst(ngr - 1), osem1).wait()

    return repack(edge_adj, idx_tab)


def kernel(node_attr, edge_adj, W1, b1, W2, b2, root, bias):
    B, N, IN_C = node_attr.shape
    D_EDGE = edge_adj.shape[-1]
    HID = W1.shape[1]
    OUT_C = root.shape[1]
    ROWS = N * N // _PACK
    CH = 1024
    NC = ROWS // CH

    import numpy as np
    u = np.arange(16)
    tab = np.zeros((_PACK * D_EDGE + 16, 16), np.int32)
    for q in range(_PACK):
        for t8 in range(D_EDGE):
            f = 128 * q + 16 * t8 + u
            tab[q * D_EDGE + t8] = f // D_EDGE
    tab[_PACK * D_EDGE] = u % D_EDGE
    for sl in range(8):
        tab[_PACK * D_EDGE + 1 + sl] = sl
    ea_p = _sc_repack(edge_adj, jnp.asarray(tab), B, N, D_EDGE)

    w1bd = jnp.kron(jnp.eye(_PACK, dtype=W1.dtype), W1)
    b1t = jnp.tile(b1, _PACK).reshape(1, _PACK * HID)
    w2t = jnp.tile(W2, (_PACK, 1))
    b2r = b2.reshape(1, IN_C * OUT_C)
    biasr = bias.reshape(1, OUT_C)

    kern = functools.partial(_tc_kernel, N=N, HID=HID, IN_C=IN_C,
                             OUT_C=OUT_C, CH=CH, NC=NC)
    out = pl.pallas_call(
        kern,
        grid=(B, NC),
        in_specs=[
            pl.BlockSpec((1, CH, _PACK * D_EDGE), lambda b, c: (b, c, 0)),
            pl.BlockSpec((1, N, IN_C), lambda b, c: (b, 0, 0)),
            pl.BlockSpec((_PACK * D_EDGE, _PACK * HID), lambda b, c: (0, 0)),
            pl.BlockSpec((1, _PACK * HID), lambda b, c: (0, 0)),
            pl.BlockSpec((_PACK * HID, IN_C * OUT_C), lambda b, c: (0, 0)),
            pl.BlockSpec((1, IN_C * OUT_C), lambda b, c: (0, 0)),
            pl.BlockSpec((IN_C, OUT_C), lambda b, c: (0, 0)),
            pl.BlockSpec((1, OUT_C), lambda b, c: (0, 0)),
        ],
        out_specs=pl.BlockSpec((1, N, OUT_C), lambda b, c: (b, 0, 0)),
        out_shape=jax.ShapeDtypeStruct((B, N, OUT_C), jnp.float32),
        scratch_shapes=[pltpu.VMEM((_PACK, _PACK * HID), jnp.float32)],
        compiler_params=pltpu.CompilerParams(
            dimension_semantics=("parallel", "arbitrary")),
    )(ea_p, node_attr, w1bd, b1t, w2t, b2r, root, biasr)
    return out


# speedup vs baseline: 2.4620x; 2.2112x over previous
import functools

import jax
import jax.numpy as jnp
from jax import lax
from jax.experimental import pallas as pl
from jax.experimental.pallas import tpu as pltpu


def _nnconv_kernel(ea0_ref, ea1_ref, na_ref, w1_ref, b1_ref, w2_ref, b2_ref,
                   root_ref, bias_ref, out_ref, hsum_ref, *, N, HID, IN_C,
                   OUT_C, CH, NC):
    c = pl.program_id(1)
    part = None
    for ref in (ea0_ref, ea1_ref):
        x = ref[0]
        h = jnp.maximum(
            jnp.dot(x, w1_ref[...], preferred_element_type=jnp.float32)
            + b1_ref[0], 0.0)
        p = jnp.sum(h.reshape(CH // N, N, HID), axis=0)
        part = p if part is None else part + p

    @pl.when(c == 0)
    def _():
        hsum_ref[...] = part

    @pl.when(c > 0)
    def _():
        hsum_ref[...] = hsum_ref[...] + part

    @pl.when(c == NC - 1)
    def _():
        ws = jnp.dot(hsum_ref[...], w2_ref[...],
                     preferred_element_type=jnp.float32) + N * b2_ref[0]
        na = na_ref[0]
        KO = IN_C * OUT_C
        R = (lax.broadcasted_iota(jnp.int32, (IN_C, KO), 1) // OUT_C ==
             lax.broadcasted_iota(jnp.int32, (IN_C, KO), 0)).astype(jnp.float32)
        S = (lax.broadcasted_iota(jnp.int32, (KO, OUT_C), 0) % OUT_C ==
             lax.broadcasted_iota(jnp.int32, (KO, OUT_C), 1)).astype(jnp.float32)
        msg = jnp.dot(jnp.dot(na, R, preferred_element_type=jnp.float32) * ws, S,
                      preferred_element_type=jnp.float32)
        out_ref[0] = msg + jnp.dot(na, root_ref[...],
                                   preferred_element_type=jnp.float32) + bias_ref[0]


def kernel(node_attr, edge_adj, W1, b1, W2, b2, root, bias):
    B, N, IN_C = node_attr.shape
    D_EDGE = edge_adj.shape[-1]
    HID = W1.shape[1]
    OUT_C = root.shape[1]
    NN = N * N
    CH = 8192
    NC = NN // (2 * CH)

    ea2 = edge_adj.reshape(B, NN, D_EDGE)
    b1r = b1.reshape(1, HID)
    b2r = b2.reshape(1, IN_C * OUT_C)
    biasr = bias.reshape(1, OUT_C)

    kern = functools.partial(_nnconv_kernel, N=N, HID=HID, IN_C=IN_C,
                             OUT_C=OUT_C, CH=CH, NC=NC)
    out = pl.pallas_call(
        kern,
        grid=(B, NC),
        in_specs=[
            pl.BlockSpec((1, CH, D_EDGE), lambda b, c: (b, 2 * c, 0)),
            pl.BlockSpec((1, CH, D_EDGE), lambda b, c: (b, 2 * c + 1, 0)),
            pl.BlockSpec((1, N, IN_C), lambda b, c: (b, 0, 0)),
            pl.BlockSpec((D_EDGE, HID), lambda b, c: (0, 0)),
            pl.BlockSpec((1, HID), lambda b, c: (0, 0)),
            pl.BlockSpec((HID, IN_C * OUT_C), lambda b, c: (0, 0)),
            pl.BlockSpec((1, IN_C * OUT_C), lambda b, c: (0, 0)),
            pl.BlockSpec((IN_C, OUT_C), lambda b, c: (0, 0)),
            pl.BlockSpec((1, OUT_C), lambda b, c: (0, 0)),
        ],
        out_specs=pl.BlockSpec((1, N, OUT_C), lambda b, c: (b, 0, 0)),
        out_shape=jax.ShapeDtypeStruct((B, N, OUT_C), jnp.float32),
        scratch_shapes=[pltpu.VMEM((N, HID), jnp.float32)],
        compiler_params=pltpu.CompilerParams(
            dimension_semantics=("parallel", "arbitrary")),
    )(ea2, ea2, node_attr, W1, b1r, W2, b2r, root, biasr)
    return out
